# baseline (speedup 1.0000x reference)
import functools

import jax
import jax.numpy as jnp
from jax import lax
from jax.experimental import pallas as pl
from jax.experimental.pallas import tpu as pltpu
from jax.experimental.pallas import tpu_sc as plsc

_BATCH = 4096
_SEQ = 200
_D = 64
_NC = 2
_NS = 16
_NW = _NC * _NS
_BPW = _BATCH // _NW
_NB = 5
_NI = 10
_LI = _NB + 2


def _sc_body(xT2, tok, posf, out5, idx_v, rows_v, tile_v, pos_v, *sems):
    gsem = sems[:_NB]
    wsem = sems[_NB : 2 * _NB]
    isem = sems[2 * _NB :]
    wid = lax.axis_index("s") * _NC + lax.axis_index("c")
    b0 = wid * _BPW
    pltpu.sync_copy(posf, pos_v)

    iota = lax.iota(jnp.int32, 16)
    ci128 = [(iota + 16 * c) * 128 for c in range(4)]
    i200 = iota * 200

    def fire_idx(s, bi):
        pltpu.make_async_copy(
            xT2.at[s, pl.ds(b0, _BPW)], idx_v.at[bi], isem[bi]
        ).start()

    def wait_idx(s, bi):
        pltpu.make_async_copy(
            xT2.at[s, pl.ds(b0, _BPW)], idx_v.at[bi], isem[bi]
        ).wait()

    def fire_gather(bi, b):
        pltpu.make_async_copy(tok.at[idx_v.at[bi]], rows_v.at[b], gsem[b]).start()

    def wait_gather(bi, b):
        pltpu.make_async_copy(tok.at[idx_v.at[bi]], rows_v.at[b], gsem[b]).wait()

    def fire_write(s, b):
        for dh in range(8):
            pltpu.make_async_copy(
                tile_v.at[b, pl.ds(1024 * dh, 1024)],
                out5.at[s, dh, wid],
                wsem[b],
            ).start()

    def wait_write(s, b):
        for dh in range(8):
            pltpu.make_async_copy(
                tile_v.at[b, pl.ds(1024 * dh, 1024)],
                out5.at[s, dh, wid],
                wsem[b],
            ).wait()

    for u in range(_NB):
        pltpu.sync_copy(xT2.at[u, pl.ds(b0, _BPW)], idx_v.at[u])
        fire_gather(u, u)
    for u in range(_NB, _LI):
        fire_idx(u, u)

    def step(t, carry):
        for u in range(_NI):
            b = u % _NB
            s = t * _NI + u
            wait_gather(u, b)
            s3 = s + _LI

            @pl.when(s3 < _SEQ)
            def _():
                fire_idx(s3, (u + _LI) % _NI)

            s2 = s + _NB

            @pl.when(s >= _NB)
            def _():
                wait_write(s - _NB, b)

            pv = [
                plsc.load_gather(pos_v, [i200 + (3200 * c + s)]) for c in range(4)
            ]
            rb = rows_v.at[b]
            tb = tile_v.at[b]

            @plsc.parallel_loop(0, _BPW, 1, unroll=4, carry=list(ci128))
            def _row(j, fidx, rb=rb, tb=tb, pv=pv):
                for c in range(4):
                    v = rb[j, pl.ds(16 * c, 16)] + pv[c]
                    plsc.store_scatter(tb, [fidx[c]], v)
                return [f + 1 for f in fidx]

            fire_write(s, b)

            @pl.when(s2 < _SEQ)
            def _():
                wait_idx(s2, (u + _NB) % _NI)
                fire_gather((u + _NB) % _NI, b)
        return carry

    lax.fori_loop(0, _SEQ // _NI, step, 0)
    for b in range(_NB):
        wait_write(_SEQ - _NB + b, b)


@jax.jit
def kernel(x, token_table, pos_table):
    xT2 = jnp.asarray(x, jnp.int32).T
    posf = pos_table.astype(jnp.float32).T.reshape(-1)
    mesh = plsc.VectorSubcoreMesh(core_axis_name="c", subcore_axis_name="s")
    f = functools.partial(
        pl.kernel,
        mesh=mesh,
        out_type=jax.ShapeDtypeStruct((_SEQ, _D // 8, _NW, 8 * _BPW), jnp.float32),
        scratch_types=[
            pltpu.VMEM((_NI, _BPW), jnp.int32),
            pltpu.VMEM((_NB, _BPW, _D), jnp.float32),
            pltpu.VMEM((_NB, _D * _BPW), jnp.float32),
            pltpu.VMEM((_D * _SEQ,), jnp.float32),
        ]
        + [pltpu.SemaphoreType.DMA] * (2 * _NB + _NI),
        compiler_params=pltpu.CompilerParams(
            use_tc_tiling_on_sc=False, needs_layout_passes=False
        ),
    )(_sc_body)
    out5 = f(xT2, token_table, posf)
    return (
        out5.reshape(_SEQ, _D // 8, _NW, 8, _BPW)
        .transpose(2, 4, 0, 1, 3)
        .reshape(_BATCH, _SEQ, _D)
    )


# speedup vs baseline: 1.0809x; 1.0809x over previous
import functools

import jax
import jax.numpy as jnp
from jax import lax
from jax.experimental import pallas as pl
from jax.experimental.pallas import tpu as pltpu
from jax.experimental.pallas import tpu_sc as plsc

_BATCH = 4096
_SEQ = 200
_D = 64
_NC = 2
_NS = 16
_NW = _NC * _NS
_BPW = _BATCH // _NW
_NB = 4
_NI = 8
_LI = _NB + 2


def _sc_body(xT2, tok2, posf, out5, idx_v, rows_v, tile_v, pos_v, *sems):
    gsem = sems[:_NB]
    wsem = sems[_NB : 2 * _NB]
    isem = sems[2 * _NB :]
    wid = lax.axis_index("s") * _NC + lax.axis_index("c")
    b0 = wid * _BPW
    pltpu.sync_copy(posf, pos_v)

    iota = lax.iota(jnp.int32, 16)
    zero16 = iota * 0
    j16 = [iota + 16 * j0 for j0 in range(8)]

    def fire_idx(s, bi):
        pltpu.make_async_copy(
            xT2.at[s, pl.ds(b0, _BPW)], idx_v.at[bi], isem[bi]
        ).start()

    def wait_idx(s, bi):
        pltpu.make_async_copy(
            xT2.at[s, pl.ds(b0, _BPW)], idx_v.at[bi], isem[bi]
        ).wait()

    def fire_gather(bi, b):
        pltpu.make_async_copy(tok2.at[idx_v.at[bi]], rows_v.at[b], gsem[b]).start()

    def wait_gather(bi, b):
        pltpu.make_async_copy(tok2.at[idx_v.at[bi]], rows_v.at[b], gsem[b]).wait()

    def fire_write(s, b):
        for dh in range(8):
            pltpu.make_async_copy(
                tile_v.at[b, pl.ds(1024 * dh, 1024)],
                out5.at[s, dh, wid],
                wsem[b],
            ).start()

    def wait_write(s, b):
        for dh in range(8):
            pltpu.make_async_copy(
                tile_v.at[b, pl.ds(1024 * dh, 1024)],
                out5.at[s, dh, wid],
                wsem[b],
            ).wait()

    for u in range(_NB):
        pltpu.sync_copy(xT2.at[u, pl.ds(b0, _BPW)], idx_v.at[u])
        fire_gather(u, u)
    for u in range(_NB, _LI):
        fire_idx(u, u)

    def step(t, carry):
        for u in range(_NI):
            b = u % _NB
            s = t * _NI + u
            wait_gather(u, b)
            s3 = s + _LI

            @pl.when(s3 < _SEQ)
            def _():
                fire_idx(s3, (u + _LI) % _NI)

            s2 = s + _NB

            @pl.when(s >= _NB)
            def _():
                wait_write(s - _NB, b)

            rb = rows_v.at[b]
            tb = tile_v.at[b]

            @plsc.parallel_loop(0, _D, 1, unroll=2)
            def _dim(d, rb=rb, tb=tb, s=s):
                pd = plsc.load_gather(pos_v, [zero16 + (d * 200 + s)])
                dv = zero16 + d
                base = d * 128
                for j0 in range(8):
                    v = plsc.load_gather(rb, [j16[j0], dv]) + pd
                    tb[pl.ds(base + 16 * j0, 16)] = v

            fire_write(s, b)

            @pl.when(s2 < _SEQ)
            def _():
                wait_idx(s2, (u + _NB) % _NI)
                fire_gather((u + _NB) % _NI, b)
        return carry

    lax.fori_loop(0, _SEQ // _NI, step, 0)
    for b in range(_NB):
        wait_write(_SEQ - _NB + b, b)


@jax.jit
def kernel(x, token_table, pos_table):
    xT2 = jnp.asarray(x, jnp.int32).T
    tok2 = jnp.pad(token_table, ((0, 0), (0, 64)))
    posf = pos_table.astype(jnp.float32).T.reshape(-1)
    mesh = plsc.VectorSubcoreMesh(core_axis_name="c", subcore_axis_name="s")
    f = functools.partial(
        pl.kernel,
        mesh=mesh,
        out_type=jax.ShapeDtypeStruct((_SEQ, _D // 8, _NW, 8 * _BPW), jnp.float32),
        scratch_types=[
            pltpu.VMEM((_NI, _BPW), jnp.int32),
            pltpu.VMEM((_NB, _BPW, 128), jnp.float32),
            pltpu.VMEM((_NB, _D * _BPW), jnp.float32),
            pltpu.VMEM((_D * _SEQ,), jnp.float32),
        ]
        + [pltpu.SemaphoreType.DMA] * (2 * _NB + _NI),
        compiler_params=pltpu.CompilerParams(
            use_tc_tiling_on_sc=False, needs_layout_passes=False
        ),
    )(_sc_body)
    out5 = f(xT2, tok2, posf)
    return (
        out5.reshape(_SEQ, _D // 8, _NW, 8, _BPW)
        .transpose(2, 4, 0, 1, 3)
        .reshape(_BATCH, _SEQ, _D)
    )
